# Initial kernel scaffold; baseline (speedup 1.0000x reference)
import jax
import jax.numpy as jnp
from jax.experimental import pallas as pl


def kernel(user_emb, entity_emb, edge_index, edge_type, mat_row, mat_col, mat_val, weight, gate1_w0, gate2_w0, gate1_w1, gate2_w1):
    raise NotImplementedError("write your pallas kernel here")


# speedup vs baseline: 6.1199x; 6.1199x over previous
import jax
import jax.numpy as jnp
from jax import lax
from jax.experimental import pallas as pl
from jax.experimental.pallas import tpu as pltpu
from jax.experimental.pallas import tpu_sc as plsc

N_USERS = 50000
N_ITEMS = 30000
N_ENT = 50000
N_REL = 16
N_EDGES = 800000
NNZ = 500000
D = 64
DH = 32
NT = 16

f32 = jnp.float32
i32 = jnp.int32

_MESH = plsc.VectorSubcoreMesh(core_axis_name="c", subcore_axis_name="s")
_SC_PARAMS = pltpu.CompilerParams(use_tc_tiling_on_sc=False)


def _zero_zbuf(zbuf, zr):
    for j in range(zr):
        zbuf[j, pl.ds(0, 16)] = jnp.zeros((16,), f32)
        zbuf[j, pl.ds(16, 16)] = jnp.zeros((16,), f32)


NA_PAD = 50176
NB_PAD = 30208
CNT_CHUNK = 2000


def _counts_body(head_hbm, mcol_hbm, inva_hbm, invb_hbm,
                 acc, idx_a, idx_b, ones, cbuf, zbuf):
    c = lax.axis_index("c")
    s = lax.axis_index("s")
    for j in range(CNT_CHUNK // 16):
        ones[pl.ds(j * 16, 16)] = jnp.full((16,), 1.0, f32)
    for j in range(98):
        zbuf[pl.ds(j * 16, 16)] = jnp.zeros((16,), f32)
    @pl.loop(0, 2)
    def _(k):
        pltpu.sync_copy(zbuf, acc.at[pl.ds(s * 3136 + k * 1568, 1568)])
    plsc.subcore_barrier()

    @pl.when(c == 0)
    def _():
        @pl.loop(0, 25)
        def _(k):
            e0 = s * 50000 + k * CNT_CHUNK
            pltpu.sync_copy(head_hbm.at[pl.ds(e0, CNT_CHUNK)], idx_a)
            pltpu.sync_copy(ones, acc.at[idx_a], add=True)

    @pl.when(c == 1)
    def _():
        nchunks = jnp.where(s < 10, 16, 15)
        @pl.loop(0, nchunks)
        def _(j):
            e0 = (s + 16 * j) * CNT_CHUNK
            pltpu.sync_copy(mcol_hbm.at[pl.ds(e0, CNT_CHUNK)], idx_b)
            pltpu.sync_copy(ones, acc.at[idx_b], add=True)

    plsc.subcore_barrier()

    @pl.when(c == 0)
    def _():
        lo = s * 3136
        pltpu.sync_copy(acc.at[pl.ds(lo, 3136)], cbuf)
        @plsc.parallel_loop(0, 196, unroll=4)
        def _(j):
            v = cbuf[pl.ds(j * 16, 16)]
            cbuf[pl.ds(j * 16, 16)] = 1.0 / jnp.maximum(v, 1.0)
        pltpu.sync_copy(cbuf, inva_hbm.at[pl.ds(lo, 3136)])

    @pl.when(c == 1)
    def _():
        lo = s * 1888
        pltpu.sync_copy(acc.at[pl.ds(lo, 1888)], cbuf.at[pl.ds(0, 1888)])
        @plsc.parallel_loop(0, 118, unroll=4)
        def _(j):
            v = cbuf[pl.ds(j * 16, 16)]
            cbuf[pl.ds(j * 16, 16)] = 1.0 / jnp.maximum(v, 1.0)
        pltpu.sync_copy(cbuf.at[pl.ds(0, 1888)], invb_hbm.at[pl.ds(lo, 1888)])


_counts_kernel = pl.kernel(
    _counts_body,
    out_type=[jax.ShapeDtypeStruct((NA_PAD,), f32),
              jax.ShapeDtypeStruct((NB_PAD,), f32)],
    mesh=_MESH,
    compiler_params=_SC_PARAMS,
    scratch_types=[
        pltpu.VMEM_SHARED((NA_PAD,), f32),
        pltpu.VMEM((CNT_CHUNK,), i32),
        pltpu.VMEM((CNT_CHUNK,), i32),
        pltpu.VMEM((CNT_CHUNK,), f32),
        pltpu.VMEM((3136,), f32),
        pltpu.VMEM((1568,), f32),
    ],
)


CA = 400
RTA = N_ENT // NT


def _pass_a_body(ent_hbm, w_hbm, tail_hbm, ety_hbm, head_hbm, out_hbm,
                 acc, w_sp, tidx, ridx, hidx, rows, wrows, wtmp, zbuf,
                 sem, sem2):
    c = lax.axis_index("c")
    s = lax.axis_index("s")

    @pl.when(s == 0)
    def _():
        pltpu.sync_copy(w_hbm.at[c], wtmp)
        pltpu.sync_copy(wtmp, w_sp)

    _zero_zbuf(zbuf, 25)
    @pl.loop(0, RTA // 25)
    def _(k):
        pltpu.sync_copy(zbuf, acc.at[pl.ds(s * RTA + k * 25, 25)])
    plsc.subcore_barrier()

    @pl.loop(0, 125)
    def _(k):
        e0 = s * 50000 + k * CA
        pltpu.sync_copy(tail_hbm.at[pl.ds(e0, CA)], tidx)
        pltpu.sync_copy(ety_hbm.at[pl.ds(e0, CA)], ridx)
        pltpu.sync_copy(head_hbm.at[pl.ds(e0, CA)], hidx)
        pltpu.async_copy(ent_hbm.at[c].at[tidx], rows, sem).wait()
        pltpu.async_copy(w_sp.at[ridx], wrows, sem2).wait()

        @plsc.parallel_loop(0, CA, unroll=8)
        def _(i):
            rows[i, pl.ds(0, 16)] = rows[i, pl.ds(0, 16)] * wrows[i, pl.ds(0, 16)]
            rows[i, pl.ds(16, 16)] = rows[i, pl.ds(16, 16)] * wrows[i, pl.ds(16, 16)]

        pltpu.sync_copy(rows, acc.at[hidx], add=True)

    plsc.subcore_barrier()
    pltpu.sync_copy(acc.at[pl.ds(s * RTA, RTA)],
                    out_hbm.at[c].at[pl.ds(s * RTA, RTA)])


_pass_a_kernel = pl.kernel(
    _pass_a_body,
    out_type=jax.ShapeDtypeStruct((2, N_ENT, DH), f32),
    mesh=_MESH,
    compiler_params=_SC_PARAMS,
    scratch_types=[
        pltpu.VMEM_SHARED((N_ENT, DH), f32),
        pltpu.VMEM_SHARED((N_REL, DH), f32),
        pltpu.VMEM((CA,), i32),
        pltpu.VMEM((CA,), i32),
        pltpu.VMEM((CA,), i32),
        pltpu.VMEM((CA, DH), f32),
        pltpu.VMEM((CA, DH), f32),
        pltpu.VMEM((N_REL, DH), f32),
        pltpu.VMEM((25, DH), f32),
        pltpu.SemaphoreType.DMA,
        pltpu.SemaphoreType.DMA,
    ],
)


CB = 2000
RTB = N_ITEMS // NT


def _pass_b_body(usr_hbm, w_hbm, mrow_hbm, mcol_hbm, out_hbm,
                 acc, gidx, sidx, rows, wvec, zbuf, sem):
    c = lax.axis_index("c")
    s = lax.axis_index("s")

    _zero_zbuf(zbuf, 25)
    @pl.loop(0, RTB // 25)
    def _(k):
        pltpu.sync_copy(zbuf, acc.at[pl.ds(s * RTB + k * 25, 25)])
    pltpu.sync_copy(w_hbm.at[c].at[pl.ds(0, 1)], wvec)
    plsc.subcore_barrier()

    w0 = wvec[0, pl.ds(0, 16)]
    w1 = wvec[0, pl.ds(16, 16)]
    nchunks = jnp.where(s < 10, 16, 15)

    @pl.loop(0, nchunks)
    def _(j):
        e0 = (s + 16 * j) * CB
        pltpu.sync_copy(mrow_hbm.at[pl.ds(e0, CB)], gidx)
        pltpu.sync_copy(mcol_hbm.at[pl.ds(e0, CB)], sidx)
        pltpu.async_copy(usr_hbm.at[c].at[gidx], rows, sem).wait()

        @plsc.parallel_loop(0, CB, unroll=8)
        def _(i):
            rows[i, pl.ds(0, 16)] = rows[i, pl.ds(0, 16)] * w0
            rows[i, pl.ds(16, 16)] = rows[i, pl.ds(16, 16)] * w1

        pltpu.sync_copy(rows, acc.at[sidx], add=True)

    plsc.subcore_barrier()
    pltpu.sync_copy(acc.at[pl.ds(s * RTB, RTB)],
                    out_hbm.at[c].at[pl.ds(s * RTB, RTB)])


_pass_b_kernel = pl.kernel(
    _pass_b_body,
    out_type=jax.ShapeDtypeStruct((2, N_ITEMS, DH), f32),
    mesh=_MESH,
    compiler_params=_SC_PARAMS,
    scratch_types=[
        pltpu.VMEM_SHARED((N_ITEMS, DH), f32),
        pltpu.VMEM((CB,), i32),
        pltpu.VMEM((CB,), i32),
        pltpu.VMEM((CB, DH), f32),
        pltpu.VMEM((1, DH), f32),
        pltpu.VMEM((25, DH), f32),
        pltpu.SemaphoreType.DMA,
    ],
)


RTC = N_USERS // NT


CC = 800


def _pass_c_body(fus_hbm, mcol_hbm, mrow_hbm, out_hbm,
                 acc, gidx, sidx, rows, zbuf, sem):
    c = lax.axis_index("c")
    s = lax.axis_index("s")

    _zero_zbuf(zbuf, 25)
    @pl.loop(0, RTC // 25)
    def _(k):
        pltpu.sync_copy(zbuf, acc.at[pl.ds(s * RTC + k * 25, 25)])
    plsc.subcore_barrier()

    nchunks = jnp.where(s < 1, 40, 39)

    @pl.loop(0, nchunks)
    def _(j):
        e0 = (s + 16 * j) * CC
        pltpu.sync_copy(mcol_hbm.at[pl.ds(e0, CC)], gidx)
        pltpu.sync_copy(mrow_hbm.at[pl.ds(e0, CC)], sidx)
        pltpu.async_copy(fus_hbm.at[c].at[gidx], rows, sem).wait()
        pltpu.sync_copy(rows, acc.at[sidx], add=True)

    plsc.subcore_barrier()
    pltpu.sync_copy(acc.at[pl.ds(s * RTC, RTC)],
                    out_hbm.at[c].at[pl.ds(s * RTC, RTC)])


_pass_c_kernel = pl.kernel(
    _pass_c_body,
    out_type=jax.ShapeDtypeStruct((2, N_USERS, DH), f32),
    mesh=_MESH,
    compiler_params=_SC_PARAMS,
    scratch_types=[
        pltpu.VMEM_SHARED((N_USERS, DH), f32),
        pltpu.VMEM((CC,), i32),
        pltpu.VMEM((CC,), i32),
        pltpu.VMEM((CC, DH), f32),
        pltpu.VMEM((25, DH), f32),
        pltpu.SemaphoreType.DMA,
    ],
)


R = 400


def _gate_body(sa_ref, sb_ref, inva_ref, invb_ref, g1t_ref, g2t_ref,
               fus_raw_ref, fusn_b_ref, fusn_full_ref):
    a = jnp.concatenate([sa_ref[0], sa_ref[1]], axis=1) * inva_ref[...]
    b = jnp.concatenate([sb_ref[0], sb_ref[1]], axis=1) * invb_ref[...]
    z = (jnp.dot(a, g1t_ref[...], preferred_element_type=f32)
         + jnp.dot(b, g2t_ref[...], preferred_element_type=f32))
    g = jax.nn.sigmoid(z)
    fus = g * a + (1.0 - g) * b
    fus_raw_ref[0] = fus[:, :DH]
    fus_raw_ref[1] = fus[:, DH:]
    n = jnp.sqrt(jnp.sum(fus * fus, axis=1, keepdims=True))
    fn = fus / jnp.maximum(n, 1e-12)
    fusn_b_ref[0] = fn[:, :DH]
    fusn_b_ref[1] = fn[:, DH:]
    fusn_full_ref[...] = fn


def _gate_tc(sumsA, sumsB, invA, invB, g1t, g2t):
    nb = N_ITEMS // R
    return pl.pallas_call(
        _gate_body,
        grid=(nb,),
        in_specs=[
            pl.BlockSpec((2, R, DH), lambda i: (0, i, 0)),
            pl.BlockSpec((2, R, DH), lambda i: (0, i, 0)),
            pl.BlockSpec((R, 1), lambda i: (i, 0)),
            pl.BlockSpec((R, 1), lambda i: (i, 0)),
            pl.BlockSpec((D, D), lambda i: (0, 0)),
            pl.BlockSpec((D, D), lambda i: (0, 0)),
        ],
        out_specs=[
            pl.BlockSpec((2, R, DH), lambda i: (0, i, 0)),
            pl.BlockSpec((2, R, DH), lambda i: (0, i, 0)),
            pl.BlockSpec((R, D), lambda i: (i, 0)),
        ],
        out_shape=[
            jax.ShapeDtypeStruct((2, N_ITEMS, DH), f32),
            jax.ShapeDtypeStruct((2, N_ITEMS, DH), f32),
            jax.ShapeDtypeStruct((N_ITEMS, D), f32),
        ],
    )(sumsA, sumsB, invA, invB, g1t, g2t)


def _norm_body(s_ref, nb_ref, nfull_ref):
    x = jnp.concatenate([s_ref[0], s_ref[1]], axis=1)
    n = jnp.sqrt(jnp.sum(x * x, axis=1, keepdims=True))
    xn = x / jnp.maximum(n, 1e-12)
    nb_ref[0] = xn[:, :DH]
    nb_ref[1] = xn[:, DH:]
    nfull_ref[...] = xn


def _norm_tc(sums, row0, nrows):
    nb = nrows // R
    return pl.pallas_call(
        _norm_body,
        grid=(nb,),
        in_specs=[pl.BlockSpec((2, R, DH), lambda i, r0=row0 // R: (0, i + r0, 0))],
        out_specs=[
            pl.BlockSpec((2, R, DH), lambda i: (0, i, 0)),
            pl.BlockSpec((R, D), lambda i: (i, 0)),
        ],
        out_shape=[
            jax.ShapeDtypeStruct((2, nrows, DH), f32),
            jax.ShapeDtypeStruct((nrows, D), f32),
        ],
    )(sums)


def kernel(user_emb, entity_emb, edge_index, edge_type, mat_row, mat_col,
           mat_val, weight, gate1_w0, gate2_w0, gate1_w1, gate2_w1):
    tail = edge_index[1]
    head = edge_index[0]

    def blocked(x):
        return jnp.stack([x[:, :DH], x[:, DH:]])

    ent_b = blocked(entity_emb)
    usr_b = blocked(user_emb)
    w_b = blocked(weight)
    gates = [(gate1_w0.T, gate2_w0.T), (gate1_w1.T, gate2_w1.T)]

    invA_p, invB_p = _counts_kernel(head, mat_col)
    invA = invA_p[:N_ITEMS].reshape(N_ITEMS, 1)
    invB = invB_p[:N_ITEMS].reshape(N_ITEMS, 1)

    entity_res = entity_emb
    user_res = user_emb

    for hop in range(2):
        g1t, g2t = gates[hop]
        sumsA = _pass_a_kernel(ent_b, w_b, tail, edge_type, head)
        sumsB = _pass_b_kernel(usr_b, w_b, mat_row, mat_col)
        fus_raw_b, fusn_b, fusn_full = _gate_tc(sumsA, sumsB, invA, invB, g1t, g2t)
        attn_b, attn_full = _norm_tc(sumsA, N_ITEMS, N_ENT - N_ITEMS)
        sumsU = _pass_c_kernel(fus_raw_b, mat_col, mat_row)
        usrn_b, usrn_full = _norm_tc(sumsU, 0, N_USERS)

        ent_b = jnp.concatenate([fusn_b, attn_b], axis=1)
        usr_b = usrn_b
        entity_res = entity_res + jnp.concatenate([fusn_full, attn_full], axis=0)
        user_res = user_res + usrn_full

    return (entity_res, user_res)
